# Initial kernel scaffold; baseline (speedup 1.0000x reference)
import jax
import jax.numpy as jnp
from jax.experimental import pallas as pl


def kernel(h, W1, b1, Wa, ba, Wb, bb, Wc, bc, Wcls, bcls):
    raise NotImplementedError("write your pallas kernel here")


# speedup vs baseline: 1.7106x; 1.7106x over previous
import jax
import jax.numpy as jnp
from jax import lax
from jax.experimental import pallas as pl


def _clam_sb_kernel(h_ref, W1_ref, b1_ref, Wa_ref, ba_ref, Wb_ref, bb_ref,
                    wc_ref, bc_ref, Wcls_ref, bcls_ref,
                    logits_ref, yprob_ref, yhat_ref, araw_ref):
    f32 = jnp.float32

    h1 = jnp.maximum(
        jnp.dot(h_ref[...], W1_ref[...], preferred_element_type=f32)
        + b1_ref[...], 0.0)

    a = jnp.tanh(
        jnp.dot(h1, Wa_ref[...], preferred_element_type=f32) + ba_ref[...])
    b = jax.nn.sigmoid(
        jnp.dot(h1, Wb_ref[...], preferred_element_type=f32) + bb_ref[...])
    ab = a * b

    A_row = lax.dot_general(
        wc_ref[...], ab,
        dimension_numbers=(((1,), (1,)), ((), ())),
        preferred_element_type=f32) + bc_ref[...]
    araw_ref[...] = A_row

    m = jnp.max(A_row, axis=1, keepdims=True)
    e = jnp.exp(A_row - m)
    A_soft = e / jnp.sum(e, axis=1, keepdims=True)

    M = jnp.dot(A_soft, h1, preferred_element_type=f32)
    logits = (jnp.dot(M, Wcls_ref[...], preferred_element_type=f32)
              + bcls_ref[...])
    logits_ref[...] = logits

    m2 = jnp.max(logits, axis=1, keepdims=True)
    e2 = jnp.exp(logits - m2)
    yprob_ref[...] = e2 / jnp.sum(e2, axis=1, keepdims=True)

    yhat_ref[...] = (logits[:, 1:2] > logits[:, 0:1]).astype(jnp.int32)


def kernel(h, W1, b1, Wa, ba, Wb, bb, Wc, bc, Wcls, bcls):
    out_shapes = (
        jax.ShapeDtypeStruct((1, 2), jnp.float32),
        jax.ShapeDtypeStruct((1, 2), jnp.float32),
        jax.ShapeDtypeStruct((1, 1), jnp.int32),
        jax.ShapeDtypeStruct((1, 77), jnp.float32),
    )
    logits, y_prob, y_hat, a_raw = pl.pallas_call(
        _clam_sb_kernel,
        out_shape=out_shapes,
    )(h, W1, b1.reshape(1, 512), Wa, ba.reshape(1, 256),
      Wb, bb.reshape(1, 256), Wc.reshape(1, 256), bc.reshape(1, 1),
      Wcls, bcls.reshape(1, 2))
    return (logits, y_prob, y_hat, a_raw)
